# baseline (speedup 1.0000x reference)
import functools

import jax
import jax.numpy as jnp
from jax import lax
from jax.experimental import pallas as pl
from jax.experimental.pallas import tpu as pltpu
from jax.experimental.pallas import tpu_sc as plsc

_N = 10000
_E = 320000
_F = 128
_H = 64
_C = 16

_ROW_BLK = 1000


def _tc1_body(x_ref, wn_ref, ws_ref, p_ref, s_ref):
    h = jnp.log(x_ref[...] + 1.0)
    p_ref[...] = jnp.dot(h, wn_ref[...], preferred_element_type=jnp.float32)
    s_ref[...] = jnp.dot(h, ws_ref[...], preferred_element_type=jnp.float32)


def _tc1(x, wn, ws):
    grid = (_N // _ROW_BLK,)
    return pl.pallas_call(
        _tc1_body,
        grid=grid,
        in_specs=[
            pl.BlockSpec((_ROW_BLK, _F), lambda i: (i, 0)),
            pl.BlockSpec((_F, _H), lambda i: (0, 0)),
            pl.BlockSpec((_F, _H), lambda i: (0, 0)),
        ],
        out_specs=[
            pl.BlockSpec((_ROW_BLK, _H), lambda i: (i, 0)),
            pl.BlockSpec((_ROW_BLK, _H), lambda i: (i, 0)),
        ],
        out_shape=[
            jax.ShapeDtypeStruct((_N, _H), jnp.float32),
            jax.ShapeDtypeStruct((_N, _H), jnp.float32),
        ],
    )(x, wn, ws)


def _tc2_body(sum_ref, cnt_ref, s1_ref, b1_ref, wn2_ref, ws2_ref,
              p2_ref, s2_ref, rc_ref):
    sum_nb = sum_ref[0] + sum_ref[1]
    cnt = cnt_ref[0, :, :1] + cnt_ref[1, :, :1]
    rc = 1.0 / jnp.maximum(cnt, 1.0)
    pre = s1_ref[...] + sum_nb * rc + b1_ref[...]
    h = jnp.maximum(pre, 0.0)
    nrm = jnp.sqrt(jnp.sum(h * h, axis=1, keepdims=True))
    h = h / (nrm + 1e-12)
    p2_ref[...] = jnp.dot(h, wn2_ref[...], preferred_element_type=jnp.float32)
    s2_ref[...] = jnp.dot(h, ws2_ref[...], preferred_element_type=jnp.float32)
    rc_ref[...] = rc


def _tc2(sum1, cnt1, s1, b1, wn2, ws2):
    grid = (_N // _ROW_BLK,)
    return pl.pallas_call(
        _tc2_body,
        grid=grid,
        in_specs=[
            pl.BlockSpec((2, _ROW_BLK, _H), lambda i: (0, i, 0)),
            pl.BlockSpec((2, _ROW_BLK, _C), lambda i: (0, i, 0)),
            pl.BlockSpec((_ROW_BLK, _H), lambda i: (i, 0)),
            pl.BlockSpec((1, _H), lambda i: (0, 0)),
            pl.BlockSpec((_H, _H), lambda i: (0, 0)),
            pl.BlockSpec((_H, _H), lambda i: (0, 0)),
        ],
        out_specs=[
            pl.BlockSpec((_ROW_BLK, _H), lambda i: (i, 0)),
            pl.BlockSpec((_ROW_BLK, _H), lambda i: (i, 0)),
            pl.BlockSpec((_ROW_BLK, 1), lambda i: (i, 0)),
        ],
        out_shape=[
            jax.ShapeDtypeStruct((_N, _H), jnp.float32),
            jax.ShapeDtypeStruct((_N, _H), jnp.float32),
            jax.ShapeDtypeStruct((_N, 1), jnp.float32),
        ],
    )(sum1, cnt1, s1, b1, wn2, ws2)


def _tc3_body(acc_ref, s2_ref, rc_ref, b2_ref, wmu_ref, bmu_ref, wvar_ref,
              bvar_ref, zl_ref, zs_ref):
    sum_nb = acc_ref[0] + acc_ref[1]
    pre = s2_ref[...] + sum_nb * rc_ref[...] + b2_ref[...]
    h = jnp.maximum(pre, 0.0)
    nrm = jnp.sqrt(jnp.sum(h * h, axis=1, keepdims=True))
    h = h / (nrm + 1e-12)
    zl_ref[...] = jnp.dot(h, wmu_ref[...], preferred_element_type=jnp.float32) + bmu_ref[...]
    zs_ref[...] = jnp.exp(
        jnp.dot(h, wvar_ref[...], preferred_element_type=jnp.float32) + bvar_ref[...])


def _tc3(acc2, s2, rc, b2, wmu, bmu, wvar, bvar):
    grid = (_N // _ROW_BLK,)
    return pl.pallas_call(
        _tc3_body,
        grid=grid,
        in_specs=[
            pl.BlockSpec((2, _ROW_BLK, _H), lambda i: (0, i, 0)),
            pl.BlockSpec((_ROW_BLK, _H), lambda i: (i, 0)),
            pl.BlockSpec((_ROW_BLK, 1), lambda i: (i, 0)),
            pl.BlockSpec((1, _H), lambda i: (0, 0)),
            pl.BlockSpec((_H, _H), lambda i: (0, 0)),
            pl.BlockSpec((1, _H), lambda i: (0, 0)),
            pl.BlockSpec((_H, _H), lambda i: (0, 0)),
            pl.BlockSpec((1, _H), lambda i: (0, 0)),
        ],
        out_specs=[
            pl.BlockSpec((_ROW_BLK, _H), lambda i: (i, 0)),
            pl.BlockSpec((_ROW_BLK, _H), lambda i: (i, 0)),
        ],
        out_shape=[
            jax.ShapeDtypeStruct((_N, _H), jnp.float32),
            jax.ShapeDtypeStruct((_N, _H), jnp.float32),
        ],
    )(acc2, s2, rc, b2, wmu, bmu, wvar, bvar)


_CHUNK = 80
_NBUF = 5


def _make_sc_segsum(d, with_cnt):
    nc, ns = 2, 16
    n_tiles = nc * ns
    edges_per_tile = _E // n_tiles
    n_chunks = edges_per_tile // _CHUNK
    rows_per_tile = (_N // ns) // 8 * 8
    tail_row0 = ns * rows_per_tile
    tail_rows = _N - tail_row0
    mesh = plsc.VectorSubcoreMesh(core_axis_name="c", subcore_axis_name="s",
                                  num_cores=nc, num_subcores=ns)

    if with_cnt:
        out_type = (jax.ShapeDtypeStruct((nc, _N, d), jnp.float32),
                    jax.ShapeDtypeStruct((nc, _N, _C), jnp.float32))
    else:
        out_type = jax.ShapeDtypeStruct((nc, _N, d), jnp.float32)

    scratch = [
        pltpu.VMEM((edges_per_tile,), jnp.int32),
        pltpu.VMEM((edges_per_tile,), jnp.int32),
        pltpu.VMEM((_NBUF, _CHUNK, d), jnp.float32),
        pltpu.VMEM_SHARED((_N, d), jnp.float32),
        [pltpu.SemaphoreType.DMA] * _NBUF,
        [pltpu.SemaphoreType.DMA] * _NBUF,
    ]
    if with_cnt:
        scratch += [
            pltpu.VMEM((_CHUNK, _C), jnp.float32),
            pltpu.VMEM_SHARED((_N, _C), jnp.float32),
            pltpu.SemaphoreType.DMA,
        ]

    @functools.partial(pl.kernel, out_type=out_type, mesh=mesh,
                       scratch_types=scratch,
                       compiler_params=pltpu.CompilerParams(
                           use_tc_tiling_on_sc=False))
    def segsum(table_hbm, edges_hbm, zeros_hbm, *rest):
        if with_cnt:
            (zeros_c_hbm, out_hbm, cnt_out_hbm,
             sidx, didx, rows, acc, gsems, ssems, ones, cacc, csem) = rest
        else:
            (out_hbm, sidx, didx, rows, acc, gsems, ssems) = rest
        c = lax.axis_index("c")
        s = lax.axis_index("s")
        wid = c * ns + s
        row0 = s * rows_per_tile
        e0 = wid * edges_per_tile
        pltpu.sync_copy(zeros_hbm.at[pl.ds(0, rows_per_tile)],
                        acc.at[pl.ds(row0, rows_per_tile)])
        pltpu.sync_copy(edges_hbm.at[pl.ds(e0, edges_per_tile)], sidx)
        pltpu.sync_copy(edges_hbm.at[pl.ds(_E + e0, edges_per_tile)], didx)
        if with_cnt:
            pltpu.sync_copy(zeros_c_hbm.at[pl.ds(0, rows_per_tile)],
                            cacc.at[pl.ds(row0, rows_per_tile)])
            for i in range(_CHUNK):
                ones[i] = jnp.ones((_C,), jnp.float32)

        @pl.when(s == 0)
        def _zero_tail():
            pltpu.sync_copy(zeros_hbm.at[pl.ds(0, tail_rows)],
                            acc.at[pl.ds(tail_row0, tail_rows)])
            if with_cnt:
                pltpu.sync_copy(zeros_c_hbm.at[pl.ds(0, tail_rows)],
                                cacc.at[pl.ds(tail_row0, tail_rows)])

        plsc.subcore_barrier()

        for b in range(_NBUF - 2):
            pltpu.async_copy(table_hbm.at[sidx.at[pl.ds(b * _CHUNK, _CHUNK)]],
                             rows.at[b], gsems[b])

        def outer(jo, carry):
            for b in range(_NBUF):
                j = jo * _NBUF + b
                dslc = didx.at[pl.ds(j * _CHUNK, _CHUNK)]
                pltpu.make_async_copy(
                    table_hbm.at[sidx.at[pl.ds(j * _CHUNK, _CHUNK)]],
                    rows.at[b], gsems[b]).wait()
                pltpu.async_copy(rows.at[b], acc.at[dslc], ssems[b], add=True)
                if with_cnt:
                    pltpu.async_copy(ones, cacc.at[dslc], csem, add=True)

                jp = j + _NBUF - 2
                bp = (b + _NBUF - 2) % _NBUF

                @pl.when(jp < n_chunks)
                def _next():
                    @pl.when(jp >= _NBUF)
                    def _wait_prev_scatter():
                        pltpu.make_async_copy(
                            rows.at[bp], acc.at[dslc], ssems[bp]).wait()

                    nslc = sidx.at[pl.ds(jp * _CHUNK, _CHUNK)]
                    pltpu.async_copy(table_hbm.at[nslc], rows.at[bp],
                                     gsems[bp])
            return carry

        lax.fori_loop(0, n_chunks // _NBUF, outer, 0)

        for b in range(_NBUF):
            pltpu.make_async_copy(rows.at[b],
                                  acc.at[didx.at[pl.ds(0, _CHUNK)]],
                                  ssems[b]).wait()
        if with_cnt:
            def drain(i, carry):
                pltpu.make_async_copy(ones,
                                      cacc.at[didx.at[pl.ds(0, _CHUNK)]],
                                      csem).wait()
                return carry
            lax.fori_loop(0, n_chunks, drain, 0)
        plsc.subcore_barrier()
        pltpu.sync_copy(acc.at[pl.ds(row0, rows_per_tile)],
                        out_hbm.at[c, pl.ds(row0, rows_per_tile)])
        if with_cnt:
            pltpu.sync_copy(cacc.at[pl.ds(row0, rows_per_tile)],
                            cnt_out_hbm.at[c, pl.ds(row0, rows_per_tile)])

        @pl.when(s == 0)
        def _copy_tail():
            pltpu.sync_copy(acc.at[pl.ds(tail_row0, tail_rows)],
                            out_hbm.at[c, pl.ds(tail_row0, tail_rows)])
            if with_cnt:
                pltpu.sync_copy(cacc.at[pl.ds(tail_row0, tail_rows)],
                                cnt_out_hbm.at[c, pl.ds(tail_row0, tail_rows)])

    return segsum


_sc_segsum = functools.cache(_make_sc_segsum)


def kernel(x, edge_index, Ws1, Wn1, b1, Ws2, Wn2, b2, Wmu, bmu, Wvar, bvar):
    e1d = edge_index.astype(jnp.int32).reshape(2 * _E)
    zh = jnp.zeros((_N // 16, _H), jnp.float32)
    zc = jnp.zeros((_N // 16, _C), jnp.float32)

    p1, s1 = _tc1(x, Wn1, Ws1)
    sum1, cnt1 = _sc_segsum(_H, True)(p1, e1d, zh, zc)
    p2, s2, rc = _tc2(sum1, cnt1, s1, b1.reshape(1, _H), Wn2, Ws2)
    sum2 = _sc_segsum(_H, False)(p2, e1d, zh)
    z_loc, z_scale = _tc3(sum2, s2, rc, b2.reshape(1, _H),
                          Wmu, bmu.reshape(1, _H), Wvar, bvar.reshape(1, _H))
    return (z_loc, z_scale)


# speedup vs baseline: 1.0441x; 1.0441x over previous
import functools

import jax
import jax.numpy as jnp
from jax import lax
from jax.experimental import pallas as pl
from jax.experimental.pallas import tpu as pltpu
from jax.experimental.pallas import tpu_sc as plsc

_N = 10000
_E = 320000
_F = 128
_H = 64
_C = 16

_ROW_BLK = 1000


def _tc1_body(x_ref, wn_ref, ws_ref, p_ref, s_ref):
    h = jnp.log(x_ref[...] + 1.0)
    p_ref[...] = jnp.dot(h, wn_ref[...], preferred_element_type=jnp.float32)
    s_ref[...] = jnp.dot(h, ws_ref[...], preferred_element_type=jnp.float32)


def _tc1(x, wn, ws):
    grid = (_N // _ROW_BLK,)
    return pl.pallas_call(
        _tc1_body,
        grid=grid,
        in_specs=[
            pl.BlockSpec((_ROW_BLK, _F), lambda i: (i, 0)),
            pl.BlockSpec((_F, _H), lambda i: (0, 0)),
            pl.BlockSpec((_F, _H), lambda i: (0, 0)),
        ],
        out_specs=[
            pl.BlockSpec((_ROW_BLK, _H), lambda i: (i, 0)),
            pl.BlockSpec((_ROW_BLK, _H), lambda i: (i, 0)),
        ],
        out_shape=[
            jax.ShapeDtypeStruct((_N, _H), jnp.float32),
            jax.ShapeDtypeStruct((_N, _H), jnp.float32),
        ],
    )(x, wn, ws)


def _tc2_body(sum_ref, cnt_ref, s1_ref, b1_ref, wn2_ref, ws2_ref,
              p2_ref, s2_ref, rc_ref):
    sum_nb = sum_ref[0] + sum_ref[1]
    cnt = cnt_ref[0, :, :1] + cnt_ref[1, :, :1]
    rc = 1.0 / jnp.maximum(cnt, 1.0)
    pre = s1_ref[...] + sum_nb * rc + b1_ref[...]
    h = jnp.maximum(pre, 0.0)
    nrm = jnp.sqrt(jnp.sum(h * h, axis=1, keepdims=True))
    h = h / (nrm + 1e-12)
    p2_ref[...] = jnp.dot(h, wn2_ref[...], preferred_element_type=jnp.float32)
    s2_ref[...] = jnp.dot(h, ws2_ref[...], preferred_element_type=jnp.float32)
    rc_ref[...] = rc


def _tc2(sum1, cnt1, s1, b1, wn2, ws2):
    grid = (_N // _ROW_BLK,)
    return pl.pallas_call(
        _tc2_body,
        grid=grid,
        in_specs=[
            pl.BlockSpec((2, _ROW_BLK, _H), lambda i: (0, i, 0)),
            pl.BlockSpec((2, _ROW_BLK, _C), lambda i: (0, i, 0)),
            pl.BlockSpec((_ROW_BLK, _H), lambda i: (i, 0)),
            pl.BlockSpec((1, _H), lambda i: (0, 0)),
            pl.BlockSpec((_H, _H), lambda i: (0, 0)),
            pl.BlockSpec((_H, _H), lambda i: (0, 0)),
        ],
        out_specs=[
            pl.BlockSpec((_ROW_BLK, _H), lambda i: (i, 0)),
            pl.BlockSpec((_ROW_BLK, _H), lambda i: (i, 0)),
            pl.BlockSpec((_ROW_BLK, 1), lambda i: (i, 0)),
        ],
        out_shape=[
            jax.ShapeDtypeStruct((_N, _H), jnp.float32),
            jax.ShapeDtypeStruct((_N, _H), jnp.float32),
            jax.ShapeDtypeStruct((_N, 1), jnp.float32),
        ],
    )(sum1, cnt1, s1, b1, wn2, ws2)


def _tc3_body(acc_ref, s2_ref, rc_ref, b2_ref, wmu_ref, bmu_ref, wvar_ref,
              bvar_ref, zl_ref, zs_ref):
    sum_nb = acc_ref[0] + acc_ref[1]
    pre = s2_ref[...] + sum_nb * rc_ref[...] + b2_ref[...]
    h = jnp.maximum(pre, 0.0)
    nrm = jnp.sqrt(jnp.sum(h * h, axis=1, keepdims=True))
    h = h / (nrm + 1e-12)
    zl_ref[...] = jnp.dot(h, wmu_ref[...], preferred_element_type=jnp.float32) + bmu_ref[...]
    zs_ref[...] = jnp.exp(
        jnp.dot(h, wvar_ref[...], preferred_element_type=jnp.float32) + bvar_ref[...])


def _tc3(acc2, s2, rc, b2, wmu, bmu, wvar, bvar):
    grid = (_N // _ROW_BLK,)
    return pl.pallas_call(
        _tc3_body,
        grid=grid,
        in_specs=[
            pl.BlockSpec((2, _ROW_BLK, _H), lambda i: (0, i, 0)),
            pl.BlockSpec((_ROW_BLK, _H), lambda i: (i, 0)),
            pl.BlockSpec((_ROW_BLK, 1), lambda i: (i, 0)),
            pl.BlockSpec((1, _H), lambda i: (0, 0)),
            pl.BlockSpec((_H, _H), lambda i: (0, 0)),
            pl.BlockSpec((1, _H), lambda i: (0, 0)),
            pl.BlockSpec((_H, _H), lambda i: (0, 0)),
            pl.BlockSpec((1, _H), lambda i: (0, 0)),
        ],
        out_specs=[
            pl.BlockSpec((_ROW_BLK, _H), lambda i: (i, 0)),
            pl.BlockSpec((_ROW_BLK, _H), lambda i: (i, 0)),
        ],
        out_shape=[
            jax.ShapeDtypeStruct((_N, _H), jnp.float32),
            jax.ShapeDtypeStruct((_N, _H), jnp.float32),
        ],
    )(acc2, s2, rc, b2, wmu, bmu, wvar, bvar)


_CHUNK = 128
_NBUF = 5


def _make_sc_segsum(d, with_cnt):
    nc, ns = 2, 16
    n_tiles = nc * ns
    edges_per_tile = _E // n_tiles
    n_chunks = -(-edges_per_tile // _CHUNK) // _NBUF * _NBUF
    padded_edges = n_chunks * _CHUNK
    n_pad16 = (padded_edges - edges_per_tile) // 16
    rows_per_tile = (_N // ns) // 8 * 8
    tail_row0 = ns * rows_per_tile
    tail_rows = _N - tail_row0
    mesh = plsc.VectorSubcoreMesh(core_axis_name="c", subcore_axis_name="s",
                                  num_cores=nc, num_subcores=ns)

    if with_cnt:
        out_type = (jax.ShapeDtypeStruct((nc, _N, d), jnp.float32),
                    jax.ShapeDtypeStruct((nc, _N, _C), jnp.float32))
    else:
        out_type = jax.ShapeDtypeStruct((nc, _N, d), jnp.float32)

    scratch = [
        pltpu.VMEM((padded_edges,), jnp.int32),
        pltpu.VMEM((padded_edges,), jnp.int32),
        pltpu.VMEM((_NBUF, _CHUNK, d), jnp.float32),
        pltpu.VMEM_SHARED((_N + 16, d), jnp.float32),
        [pltpu.SemaphoreType.DMA] * _NBUF,
    ]
    if with_cnt:
        scratch += [
            pltpu.VMEM((_CHUNK, _C), jnp.float32),
            pltpu.VMEM_SHARED((_N + 16, _C), jnp.float32),
        ]

    @functools.partial(pl.kernel, out_type=out_type, mesh=mesh,
                       scratch_types=scratch,
                       compiler_params=pltpu.CompilerParams(
                           use_tc_tiling_on_sc=False))
    def segsum(table_hbm, edges_hbm, zeros_hbm, *rest):
        if with_cnt:
            (zeros_c_hbm, out_hbm, cnt_out_hbm,
             sidx, didx, rows, acc, gsems, ones, cacc) = rest
        else:
            (out_hbm, sidx, didx, rows, acc, gsems) = rest
        c = lax.axis_index("c")
        s = lax.axis_index("s")
        wid = c * ns + s
        row0 = s * rows_per_tile
        e0 = wid * edges_per_tile
        pltpu.sync_copy(zeros_hbm.at[pl.ds(0, rows_per_tile)],
                        acc.at[pl.ds(row0, rows_per_tile)])
        pltpu.sync_copy(edges_hbm.at[pl.ds(e0, edges_per_tile)],
                        sidx.at[pl.ds(0, edges_per_tile)])
        pltpu.sync_copy(edges_hbm.at[pl.ds(_E + e0, edges_per_tile)],
                        didx.at[pl.ds(0, edges_per_tile)])
        for k in range(n_pad16):
            off = edges_per_tile + k * 16
            sidx[pl.ds(off, 16)] = jnp.zeros((16,), jnp.int32)
            didx[pl.ds(off, 16)] = jnp.full((16,), _N, jnp.int32)
        if with_cnt:
            pltpu.sync_copy(zeros_c_hbm.at[pl.ds(0, rows_per_tile)],
                            cacc.at[pl.ds(row0, rows_per_tile)])
            for i in range(_CHUNK):
                ones[i] = jnp.ones((_C,), jnp.float32)

        @pl.when(s == 0)
        def _zero_tail():
            pltpu.sync_copy(zeros_hbm.at[pl.ds(0, tail_rows)],
                            acc.at[pl.ds(tail_row0, tail_rows)])
            if with_cnt:
                pltpu.sync_copy(zeros_c_hbm.at[pl.ds(0, tail_rows)],
                                cacc.at[pl.ds(tail_row0, tail_rows)])

        plsc.subcore_barrier()

        for b in range(_NBUF):
            pltpu.async_copy(table_hbm.at[sidx.at[pl.ds(b * _CHUNK, _CHUNK)]],
                             rows.at[b], gsems[b])

        def outer(jo, carry):
            for b in range(_NBUF):
                j = jo * _NBUF + b
                dslc = didx.at[pl.ds(j * _CHUNK, _CHUNK)]
                pltpu.make_async_copy(
                    table_hbm.at[sidx.at[pl.ds(j * _CHUNK, _CHUNK)]],
                    rows.at[b], gsems[b]).wait()
                pltpu.sync_copy(rows.at[b], acc.at[dslc], add=True)
                if with_cnt:
                    pltpu.sync_copy(ones, cacc.at[dslc], add=True)

                @pl.when(j + _NBUF < n_chunks)
                def _next():
                    nslc = sidx.at[pl.ds((j + _NBUF) * _CHUNK, _CHUNK)]
                    pltpu.async_copy(table_hbm.at[nslc], rows.at[b], gsems[b])
            return carry

        lax.fori_loop(0, n_chunks // _NBUF, outer, 0)
        plsc.subcore_barrier()
        pltpu.sync_copy(acc.at[pl.ds(row0, rows_per_tile)],
                        out_hbm.at[c, pl.ds(row0, rows_per_tile)])
        if with_cnt:
            pltpu.sync_copy(cacc.at[pl.ds(row0, rows_per_tile)],
                            cnt_out_hbm.at[c, pl.ds(row0, rows_per_tile)])

        @pl.when(s == 0)
        def _copy_tail():
            pltpu.sync_copy(acc.at[pl.ds(tail_row0, tail_rows)],
                            out_hbm.at[c, pl.ds(tail_row0, tail_rows)])
            if with_cnt:
                pltpu.sync_copy(cacc.at[pl.ds(tail_row0, tail_rows)],
                                cnt_out_hbm.at[c, pl.ds(tail_row0, tail_rows)])

    return segsum


_sc_segsum = functools.cache(_make_sc_segsum)


def kernel(x, edge_index, Ws1, Wn1, b1, Ws2, Wn2, b2, Wmu, bmu, Wvar, bvar):
    e1d = edge_index.astype(jnp.int32).reshape(2 * _E)
    zh = jnp.zeros((_N // 16, _H), jnp.float32)
    zc = jnp.zeros((_N // 16, _C), jnp.float32)

    p1, s1 = _tc1(x, Wn1, Ws1)
    sum1, cnt1 = _sc_segsum(_H, True)(p1, e1d, zh, zc)
    p2, s2, rc = _tc2(sum1, cnt1, s1, b1.reshape(1, _H), Wn2, Ws2)
    sum2 = _sc_segsum(_H, False)(p2, e1d, zh)
    z_loc, z_scale = _tc3(sum2, s2, rc, b2.reshape(1, _H),
                          Wmu, bmu.reshape(1, _H), Wvar, bvar.reshape(1, _H))
    return (z_loc, z_scale)
